# initial kernel scaffold (unmeasured)
import jax
import jax.numpy as jnp
from jax import lax
from jax.experimental import pallas as pl
from jax.experimental.pallas import tpu as pltpu

N_DEV = 32
M = 4096
N_OUT = 2048
CHUNK = M // N_DEV


def _ar_body(p_ref, out_ref, send_buf, recv_buf, send_sem, recv_sem, credit_sem):
    d = lax.axis_index("i")
    left = lax.rem(d - 1 + N_DEV, N_DEV)
    right = lax.rem(d + 1, N_DEV)

    barrier = pltpu.get_barrier_semaphore()
    pl.semaphore_signal(
        barrier, inc=1, device_id=(left,), device_id_type=pl.DeviceIdType.MESH
    )
    pl.semaphore_signal(
        barrier, inc=1, device_id=(right,), device_id_type=pl.DeviceIdType.MESH
    )
    pl.semaphore_wait(barrier, 2)

    out_ref[...] = p_ref[...]

    def step(send_idx, recv_idx, accumulate, first):
        send_buf[...] = out_ref[pl.ds(send_idx * CHUNK, CHUNK), :]
        if not first:
            pl.semaphore_wait(credit_sem, 1)
        rdma = pltpu.make_async_remote_copy(
            src_ref=send_buf,
            dst_ref=recv_buf,
            send_sem=send_sem,
            recv_sem=recv_sem,
            device_id=(right,),
            device_id_type=pl.DeviceIdType.MESH,
        )
        rdma.start()
        rdma.wait()
        row = pl.ds(recv_idx * CHUNK, CHUNK)
        if accumulate:
            out_ref[row, :] = out_ref[row, :] + recv_buf[...]
        else:
            out_ref[row, :] = recv_buf[...]
        pl.semaphore_signal(
            credit_sem, inc=1, device_id=(left,), device_id_type=pl.DeviceIdType.MESH
        )

    for s in range(N_DEV - 1):
        send_idx = lax.rem(d - s + 2 * N_DEV, N_DEV)
        recv_idx = lax.rem(d - s - 1 + 2 * N_DEV, N_DEV)
        step(send_idx, recv_idx, accumulate=True, first=(s == 0))

    for s in range(N_DEV - 1):
        send_idx = lax.rem(d + 1 - s + 2 * N_DEV, N_DEV)
        recv_idx = lax.rem(d - s + 2 * N_DEV, N_DEV)
        step(send_idx, recv_idx, accumulate=False, first=False)

    pl.semaphore_wait(credit_sem, 1)


def kernel(x, w_mat):
    partial = jnp.dot(x, w_mat, preferred_element_type=jnp.float32)
    y = pl.pallas_call(
        _ar_body,
        out_shape=jax.ShapeDtypeStruct((M, N_OUT), jnp.float32),
        in_specs=[pl.BlockSpec(memory_space=pltpu.VMEM)],
        out_specs=pl.BlockSpec(memory_space=pltpu.VMEM),
        scratch_shapes=[
            pltpu.VMEM((CHUNK, N_OUT), jnp.float32),
            pltpu.VMEM((CHUNK, N_OUT), jnp.float32),
            pltpu.SemaphoreType.DMA,
            pltpu.SemaphoreType.DMA,
            pltpu.SemaphoreType.REGULAR,
        ],
        compiler_params=pltpu.CompilerParams(collective_id=0),
    )(partial)

    y = jnp.maximum(y, 0.0)
    amax = jnp.max(y)
    scale = amax / 448.0
    q = (y / scale).astype(jnp.float8_e4m3fn)
    return q.astype(jnp.float32) * scale


# baseline (device time: 1194659 ns/iter reference)
import jax
import jax.numpy as jnp
from jax import lax
from jax.experimental import pallas as pl
from jax.experimental.pallas import tpu as pltpu

N_DEV = 32
M = 4096
N_OUT = 2048
CHUNK = M // N_DEV


def _row(idx):
    return (pl.ds(idx * CHUNK, CHUNK), slice(None))


def _ar_body(
    p_ref,
    out_ref,
    send_buf,
    recv_buf,
    pchunk_buf,
    copy_sem,
    send_sem,
    recv_sem,
    credit_sem,
):
    d = lax.axis_index("i")
    left = lax.rem(d - 1 + N_DEV, N_DEV)
    right = lax.rem(d + 1, N_DEV)

    cp = pltpu.make_async_copy(p_ref.at[_row(d)], send_buf, copy_sem)
    cp.start()

    barrier = pltpu.get_barrier_semaphore()
    pl.semaphore_signal(
        barrier, inc=1, device_id=(left,), device_id_type=pl.DeviceIdType.MESH
    )
    pl.semaphore_signal(
        barrier, inc=1, device_id=(right,), device_id_type=pl.DeviceIdType.MESH
    )
    pl.semaphore_wait(barrier, 2)
    cp.wait()

    def ring_send():
        rdma = pltpu.make_async_remote_copy(
            src_ref=send_buf,
            dst_ref=recv_buf,
            send_sem=send_sem,
            recv_sem=recv_sem,
            device_id=(right,),
            device_id_type=pl.DeviceIdType.MESH,
        )
        rdma.start()
        return rdma

    def credit_to_left():
        pl.semaphore_signal(
            credit_sem,
            inc=1,
            device_id=(left,),
            device_id_type=pl.DeviceIdType.MESH,
        )

    for s in range(N_DEV - 1):
        if s > 0:
            pl.semaphore_wait(credit_sem, 1)
        rdma = ring_send()
        recv_idx = lax.rem(d - s - 1 + 2 * N_DEV, N_DEV)
        cp = pltpu.make_async_copy(p_ref.at[_row(recv_idx)], pchunk_buf, copy_sem)
        cp.start()
        rdma.wait()
        cp.wait()
        send_buf[...] = recv_buf[...] + pchunk_buf[...]
        credit_to_left()

    own_idx = lax.rem(d + 1, N_DEV)
    cp = pltpu.make_async_copy(send_buf, out_ref.at[_row(own_idx)], copy_sem)
    cp.start()
    cp.wait()
    for s in range(N_DEV - 1):
        pl.semaphore_wait(credit_sem, 1)
        rdma = ring_send()
        rdma.wait()
        send_buf[...] = recv_buf[...]
        credit_to_left()
        recv_idx = lax.rem(d - s + 2 * N_DEV, N_DEV)
        cp = pltpu.make_async_copy(send_buf, out_ref.at[_row(recv_idx)], copy_sem)
        cp.start()
        cp.wait()

    pl.semaphore_wait(credit_sem, 1)


def kernel(x, w_mat):
    partial = jnp.dot(x, w_mat, preferred_element_type=jnp.float32)
    y = pl.pallas_call(
        _ar_body,
        out_shape=jax.ShapeDtypeStruct((M, N_OUT), jnp.float32),
        in_specs=[pl.BlockSpec(memory_space=pl.ANY)],
        out_specs=pl.BlockSpec(memory_space=pl.ANY),
        scratch_shapes=[
            pltpu.VMEM((CHUNK, N_OUT), jnp.float32),
            pltpu.VMEM((CHUNK, N_OUT), jnp.float32),
            pltpu.VMEM((CHUNK, N_OUT), jnp.float32),
            pltpu.SemaphoreType.DMA,
            pltpu.SemaphoreType.DMA,
            pltpu.SemaphoreType.DMA,
            pltpu.SemaphoreType.REGULAR,
        ],
        compiler_params=pltpu.CompilerParams(collective_id=0),
    )(partial)

    y = jnp.maximum(y, 0.0)
    amax = jnp.max(y)
    scale = amax / 448.0
    z = y / scale
    u = jax.lax.bitcast_convert_type(z, jnp.uint32)
    u = (u + jnp.uint32(0x7FFFF) + ((u >> 20) & jnp.uint32(1))) & jnp.uint32(
        0xFFF00000
    )
    zq = jax.lax.bitcast_convert_type(u, jnp.float32)
    zq = jnp.minimum(zq, 448.0)
    zq = jnp.where(z < 2.0**-6, jnp.round(z * 512.0) * (1.0 / 512.0), zq)
    return zq * scale
